# baseline (speedup 1.0000x reference)
import functools
import math

import jax
import jax.numpy as jnp
import numpy as np
from jax import lax
from jax.experimental import pallas as pl
from jax.experimental.pallas import tpu as pltpu
from jax.experimental.pallas import tpu_sc as plsc

H = 16
Dh = 16
_SC = 1.0 / math.sqrt(Dh)
_BNS = 1.0 / math.sqrt(1.0 + 1e-5)
NPAD = 5120
NROW = 5120


def _cell_body(ge_ref, W1_ref, v1_ref, W2_ref, v2_ref, Wc_ref, bc_ref,
               cell_ref, proj_ref):
    h = jnp.dot(ge_ref[...], W1_ref[...], preferred_element_type=jnp.float32)
    h = jnp.maximum(h + v1_ref[0:1, :], 0.0) * v1_ref[1:2, :] + v1_ref[2:3, :]
    c = jnp.dot(h, W2_ref[...], preferred_element_type=jnp.float32)
    c = jnp.maximum(c + v2_ref[0:1, :], 0.0) * v2_ref[1:2, :] + v2_ref[2:3, :]
    cell_ref[...] = c
    proj_ref[...] = jnp.dot(c, Wc_ref[...], preferred_element_type=jnp.float32) + bc_ref[...]


def _cell_encoder(ge_pad, W1, v1, W2, v2, Wcat, bcat):
    n = ge_pad.shape[0]
    blk = 1024
    grid = n // blk
    return pl.pallas_call(
        _cell_body,
        grid=(grid,),
        in_specs=[
            pl.BlockSpec((blk, 512), lambda i: (i, 0)),
            pl.BlockSpec((512, 1024), lambda i: (0, 0)),
            pl.BlockSpec((3, 1024), lambda i: (0, 0)),
            pl.BlockSpec((1024, 256), lambda i: (0, 0)),
            pl.BlockSpec((3, 256), lambda i: (0, 0)),
            pl.BlockSpec((256, 1280), lambda i: (0, 0)),
            pl.BlockSpec((1, 1280), lambda i: (0, 0)),
        ],
        out_specs=[
            pl.BlockSpec((blk, 256), lambda i: (i, 0)),
            pl.BlockSpec((blk, 1280), lambda i: (i, 0)),
        ],
        out_shape=[
            jax.ShapeDtypeStruct((n, 256), jnp.float32),
            jax.ShapeDtypeStruct((n, 1280), jnp.float32),
        ],
    )(ge_pad, W1, v1, W2, v2, Wcat, bcat)


def _conv_body(x_ref, w1_ref, w2_ref, v_ref, out_ref):
    x = x_ref[:, 0]
    chans1 = []
    for o in range(2):
        y = (w1_ref[o, 0] * x[0:879] + w1_ref[o, 1] * x[1:880]
             + w1_ref[o, 2] * x[2:881] + w1_ref[o, 3])
        y = jnp.maximum(y, 0.0)
        y = y.reshape(293, 3, 8, 128).max(axis=1)
        chans1.append(y)
    chans2 = []
    for o in range(4):
        z = w2_ref[o, 6]
        for i in range(2):
            z = z + (w2_ref[o, 3 * i] * chans1[i][0:291]
                     + w2_ref[o, 3 * i + 1] * chans1[i][1:292]
                     + w2_ref[o, 3 * i + 2] * chans1[i][2:293])
        z = jnp.maximum(z, 0.0) * v_ref[o, 0] + v_ref[o, 1]
        z = z.reshape(97, 3, 8, 128).max(axis=1)
        chans2.append(z)
    out_ref[...] = jnp.concatenate(chans2, axis=0)[:, None]


def _drug_conv(fpT3, w1s, w2s, v2s):
    return pl.pallas_call(
        _conv_body,
        grid=(5,),
        in_specs=[
            pl.BlockSpec((881, 1, 8, 128), lambda i: (0, i, 0, 0)),
            pl.BlockSpec(memory_space=pltpu.SMEM),
            pl.BlockSpec(memory_space=pltpu.SMEM),
            pl.BlockSpec(memory_space=pltpu.SMEM),
        ],
        out_specs=pl.BlockSpec((388, 1, 8, 128), lambda i: (0, i, 0, 0)),
        out_shape=jax.ShapeDtypeStruct((388, 5, 8, 128), jnp.float32),
    )(fpT3, w1s, w2s, v2s)


def _drugfc_body(x_ref, W1_ref, v1_ref, W2_ref, v2_ref, Wc_ref, bc_ref,
                 drug_ref, proj_ref):
    x = x_ref[...]
    h = jax.lax.dot_general(x, W1_ref[...], (((0,), (0,)), ((), ())),
                            preferred_element_type=jnp.float32)
    h = jnp.maximum(h + v1_ref[0:1, :], 0.0) * v1_ref[1:2, :] + v1_ref[2:3, :]
    d = jnp.dot(h, W2_ref[...], preferred_element_type=jnp.float32)
    d = jnp.maximum(d + v2_ref[0:1, :], 0.0) * v2_ref[1:2, :] + v2_ref[2:3, :]
    drug_ref[...] = d
    proj_ref[...] = jnp.dot(d, Wc_ref[...], preferred_element_type=jnp.float32) + bc_ref[...]


def _drug_fc(x388, W1, v1, W2, v2, Wcat, bcat):
    n = x388.shape[1]
    blk = 1024
    return pl.pallas_call(
        _drugfc_body,
        grid=(n // blk,),
        in_specs=[
            pl.BlockSpec((388, blk), lambda i: (0, i)),
            pl.BlockSpec((388, 800), lambda i: (0, 0)),
            pl.BlockSpec((3, 800), lambda i: (0, 0)),
            pl.BlockSpec((800, 256), lambda i: (0, 0)),
            pl.BlockSpec((3, 256), lambda i: (0, 0)),
            pl.BlockSpec((256, 768), lambda i: (0, 0)),
            pl.BlockSpec((1, 768), lambda i: (0, 0)),
        ],
        out_specs=[
            pl.BlockSpec((blk, 256), lambda i: (i, 0)),
            pl.BlockSpec((blk, 768), lambda i: (i, 0)),
        ],
        out_shape=[
            jax.ShapeDtypeStruct((n, 256), jnp.float32),
            jax.ShapeDtypeStruct((n, 768), jnp.float32),
        ],
    )(x388, W1, v1, W2, v2, Wcat, bcat)


def _outproj_body(p_ref, s_ref, base_ref, R_ref, Wo_ref, bo_ref, a_ref, out_ref):
    ps = p_ref[0] + p_ref[1]
    den = jnp.dot(s_ref[0] + s_ref[1], R_ref[...],
                  preferred_element_type=jnp.float32) + 1e-16
    agg = ps / den
    g = agg * 0.5 * (1.0 + jax.lax.erf(agg * (1.0 / math.sqrt(2.0))))
    o = jnp.dot(g, Wo_ref[...], preferred_element_type=jnp.float32) + bo_ref[...]
    a = a_ref[0]
    out_ref[...] = a * o + (1.0 - a) * base_ref[...]


def _outproj(parts, s_parts, base, R, Wo, bo, a_vec):
    n = parts.shape[1]
    return pl.pallas_call(
        _outproj_body,
        grid=(1,),
        in_specs=[
            pl.BlockSpec((2, n, 256), lambda i: (0, 0, 0)),
            pl.BlockSpec((2, n, 16), lambda i: (0, 0, 0)),
            pl.BlockSpec((n, 256), lambda i: (0, 0)),
            pl.BlockSpec((16, 256), lambda i: (0, 0)),
            pl.BlockSpec((256, 256), lambda i: (0, 0)),
            pl.BlockSpec((1, 256), lambda i: (0, 0)),
            pl.BlockSpec(memory_space=pltpu.SMEM),
        ],
        out_specs=pl.BlockSpec((n, 256), lambda i: (0, 0)),
        out_shape=jax.ShapeDtypeStruct((n, 256), jnp.float32),
    )(parts, s_parts, base, R, Wo, bo, a_vec)


def _pair_body(L_ref, Rr_ref, W1t_ref, W1b_ref, b1_ref, W2_ref, b2_ref,
               W3_ref, b3_ref, out_ref):
    h = (jnp.dot(L_ref[...], W1t_ref[...], preferred_element_type=jnp.float32)
         + jnp.dot(Rr_ref[...], W1b_ref[...], preferred_element_type=jnp.float32)
         + b1_ref[...])
    h = jnp.maximum(h, 0.0)
    h = jnp.maximum(jnp.dot(h, W2_ref[...], preferred_element_type=jnp.float32)
                    + b2_ref[...], 0.0)
    z = jnp.dot(h, W3_ref[...], preferred_element_type=jnp.float32) + b3_ref[...]
    out_ref[...] = jax.nn.sigmoid(z)


def _pair_mlp(L, Rg, W1t, W1b, b1, W2, b2, W3p, b3p):
    n = L.shape[0]
    blk = 1024
    return pl.pallas_call(
        _pair_body,
        grid=(n // blk,),
        in_specs=[
            pl.BlockSpec((blk, 256), lambda i: (i, 0)),
            pl.BlockSpec((blk, 256), lambda i: (i, 0)),
            pl.BlockSpec((256, 256), lambda i: (0, 0)),
            pl.BlockSpec((256, 256), lambda i: (0, 0)),
            pl.BlockSpec((1, 256), lambda i: (0, 0)),
            pl.BlockSpec((256, 128), lambda i: (0, 0)),
            pl.BlockSpec((1, 128), lambda i: (0, 0)),
            pl.BlockSpec((128, 128), lambda i: (0, 0)),
            pl.BlockSpec((1, 128), lambda i: (0, 0)),
        ],
        out_specs=pl.BlockSpec((blk, 128), lambda i: (i, 0)),
        out_shape=jax.ShapeDtypeStruct((n, 128), jnp.float32),
    )(L, Rg, W1t, W1b, b1, W2, b2, W3p, b3p)


def _headT(w):
    if w.ndim == 1:
        return w.reshape(H, Dh).T.reshape(256)
    return w.reshape(-1, H, Dh).transpose(0, 2, 1).reshape(-1, 256)


def _fuse_tables(q, xside):
    t = 'c' if xside == 'c' else 'd'
    Wq, bq = q['Wq_' + t], q['bq_' + t]
    Wk, bk = q['Wk_' + t], q['bk_' + t]
    Wv, bv = q['Wv_' + t], q['bv_' + t]
    cols_w = [_headT(Wq)]
    cols_b = [_headT(bq)]
    etypes = ['cc', 'cd'] if xside == 'c' else ['dd']
    for e in etypes:
        scale = (q['p_' + e] * _SC)[None, :, None]
        Wkt = jnp.einsum('ihd,hde->ihe', Wk.reshape(256, H, Dh), q['a_' + e]) * scale
        bkt = jnp.einsum('hd,hde->he', bk.reshape(H, Dh), q['a_' + e]) * scale[0]
        Wvt = jnp.einsum('ihd,hde->ihe', Wv.reshape(256, H, Dh), q['m_' + e])
        bvt = jnp.einsum('hd,hde->he', bv.reshape(H, Dh), q['m_' + e])
        cols_w += [Wkt.transpose(0, 2, 1).reshape(256, 256),
                   Wvt.transpose(0, 2, 1).reshape(256, 256)]
        cols_b += [bkt.T.reshape(256), bvt.T.reshape(256)]
    return jnp.concatenate(cols_w, axis=1), jnp.concatenate(cols_b)[None, :]


def _edge_msg(qt, ktt, vtt, src, dst):
    lg = (qt[dst].reshape(-1, Dh, H) * ktt[src].reshape(-1, Dh, H)).sum(axis=1)
    e = jnp.exp(lg)
    msg = vtt[src].reshape(-1, Dh, H) * e[:, None, :]
    return jnp.concatenate([msg.reshape(-1, 256), e], axis=1)


def _edge_jnp(msgs, dst, num):
    seg = jax.ops.segment_sum(msgs, dst, num_segments=num)
    return seg[:, :256], seg[:, 256:]


def _gather_sc_body(xc_ref, xd_ref, ci_ref, di_ref, L_out, R_out,
                    idxv, rows):
    cid = lax.axis_index("c")
    sid = lax.axis_index("s")
    wid = sid * 2 + cid
    for k in range(2):
        base = pl.multiple_of(wid * 256 + k * 128, 128)
        pltpu.sync_copy(ci_ref.at[pl.ds(base, 128)], idxv)
        pltpu.sync_copy(xc_ref.at[idxv], rows)
        pltpu.sync_copy(rows, L_out.at[pl.ds(base, 128)])
        pltpu.sync_copy(di_ref.at[pl.ds(base, 128)], idxv)
        pltpu.sync_copy(xd_ref.at[idxv], rows)
        pltpu.sync_copy(rows, R_out.at[pl.ds(base, 128)])


def _gather_sc(xc, xd, ci, di):
    f32 = jnp.float32
    mesh = plsc.VectorSubcoreMesh(core_axis_name="c", subcore_axis_name="s",
                                  num_cores=2, num_subcores=16)
    fn = pl.kernel(
        _gather_sc_body,
        out_type=[
            jax.ShapeDtypeStruct((8192, 256), f32),
            jax.ShapeDtypeStruct((8192, 256), f32),
        ],
        mesh=mesh,
        scratch_types=[
            pltpu.VMEM((128,), jnp.int32),
            pltpu.VMEM((128, 256), f32),
        ],
    )
    return fn(xc, xd, ci, di)


def kernel(gene_expression_feature, CNV_feature, cell_edge_idx,
           fingerprint_feature, drug_edge_index, train_pair_mask,
           cell_drug_edge_idx, params):
    p = params
    q = p['hgt'][1]
    f32 = jnp.float32

    s1 = (p['cls_g1'] * _BNS)[None, :]
    v1 = jnp.concatenate([p['cls_b1'][None, :], s1, p['cls_bt1'][None, :]], 0)
    s2 = (p['cls_g2'] * _BNS)[None, :]
    v2 = jnp.concatenate([p['cls_b2'][None, :], s2, p['cls_bt2'][None, :]], 0)
    Wcat_c, bcat_c = _fuse_tables(q, 'c')
    Wcat_d, bcat_d = _fuse_tables(q, 'd')

    s1c = p['fp_g1'] * _BNS
    t1c = p['fp_bt1']
    w1s = jnp.concatenate([p['fp_cw1'][:, 0, :], p['fp_cb1'][:, None]], 1)
    w2f = p['fp_cw2'] * s1c[None, :, None]
    b2f = p['fp_cb2'] + (p['fp_cw2'] * t1c[None, :, None]).sum(axis=(1, 2))
    w2s = jnp.concatenate([w2f.reshape(4, 6), b2f[:, None]], 1)
    v2s = jnp.stack([p['fp_g2'] * _BNS, p['fp_bt2']], axis=1)
    s3 = (p['fp_g3'] * _BNS)[None, :]
    v3 = jnp.concatenate([p['fp_b1'][None, :], s3, p['fp_bt3'][None, :]], 0)
    s4 = (p['fp_g4'] * _BNS)[None, :]
    v4 = jnp.concatenate([p['fp_b2'][None, :], s4, p['fp_bt4'][None, :]], 0)

    Wo_c = _headT(q['Wo_c'].T).T
    Wo_d = _headT(q['Wo_d'].T).T
    Rrep = jnp.tile(jnp.eye(16, dtype=f32), (1, 16))
    ac = jax.nn.sigmoid(q['skip_c'])[None]
    ad = jax.nn.sigmoid(q['skip_d'])[None]

    ge_pad = jnp.pad(gene_expression_feature, ((0, NROW - 5000), (0, 0)))
    cell, projc = _cell_encoder(ge_pad, p['cls_W1'], v1, p['cls_W2'], v2,
                                Wcat_c, bcat_c)

    fpT = jnp.pad(fingerprint_feature, ((0, NROW - 5000), (0, 0))).T
    fpT3 = fpT.reshape(881, 5, 8, 128)
    x388 = _drug_conv(fpT3, w1s, w2s, v2s).reshape(388, NROW)
    drug, projd = _drug_fc(x388, p['fp_W1'], v3, p['fp_W2'], v4, Wcat_d, bcat_d)

    qc_t, ktcc, vtcc, ktcd, vtcd = [projc[:, i * 256:(i + 1) * 256] for i in range(5)]
    qd_t, ktdd, vtdd = [projd[:, i * 256:(i + 1) * 256] for i in range(3)]

    mc = _edge_msg(qc_t, ktcc, vtcc, cell_edge_idx[0], cell_edge_idx[1])
    md1 = _edge_msg(qd_t, ktdd, vtdd, drug_edge_index[0], drug_edge_index[1])
    md2 = _edge_msg(qd_t, ktcd, vtcd, cell_drug_edge_idx[0], cell_drug_edge_idx[1])
    aggc, s_c = _edge_jnp(mc, cell_edge_idx[1], NPAD)
    aggd, s_d = _edge_jnp(jnp.concatenate([md1, md2], axis=0),
                          jnp.concatenate([drug_edge_index[1],
                                           cell_drug_edge_idx[1]], axis=0), NPAD)
    z256 = jnp.zeros((NPAD, 256), f32)
    z16 = jnp.zeros((NPAD, 16), f32)
    partc = jnp.stack([aggc, z256])
    spartc = jnp.stack([s_c, z16])
    partd = jnp.stack([aggd, z256])
    spartd = jnp.stack([s_d, z16])

    xc = _outproj(partc, spartc, cell, Rrep, Wo_c, q['bo_c'][None, :], ac)
    xd = _outproj(partd, spartd, drug, Rrep, Wo_d, q['bo_d'][None, :], ad)

    ci = train_pair_mask[:, 1]
    di = train_pair_mask[:, 0]
    L, Rg = _gather_sc(xc, xd, ci, di)

    W3p = jnp.pad(p['cmb_W3'], ((0, 0), (0, 127)))
    b3p = jnp.pad(p['cmb_b3'], (0, 127))[None, :]
    res = _pair_mlp(L, Rg, p['cmb_W1'][:256], p['cmb_W1'][256:],
                    p['cmb_b1'][None, :], p['cmb_W2'], p['cmb_b2'][None, :],
                    W3p, b3p)
    return res[:, 0]


# speedup vs baseline: 1.0984x; 1.0984x over previous
import functools
import math

import jax
import jax.numpy as jnp
import numpy as np
from jax import lax
from jax.experimental import pallas as pl
from jax.experimental.pallas import tpu as pltpu
from jax.experimental.pallas import tpu_sc as plsc

H = 16
Dh = 16
_SC = 1.0 / math.sqrt(Dh)
_BNS = 1.0 / math.sqrt(1.0 + 1e-5)
NPAD = 5120
NROW = 5120


def _cell_body(ge_ref, W1_ref, v1_ref, W2_ref, v2_ref, Wc_ref, bc_ref,
               cell_ref, proj_ref):
    h = jnp.dot(ge_ref[...], W1_ref[...], preferred_element_type=jnp.float32)
    h = jnp.maximum(h + v1_ref[0:1, :], 0.0) * v1_ref[1:2, :] + v1_ref[2:3, :]
    c = jnp.dot(h, W2_ref[...], preferred_element_type=jnp.float32)
    c = jnp.maximum(c + v2_ref[0:1, :], 0.0) * v2_ref[1:2, :] + v2_ref[2:3, :]
    cell_ref[...] = c
    proj_ref[...] = jnp.dot(c, Wc_ref[...], preferred_element_type=jnp.float32) + bc_ref[...]


def _cell_encoder(ge_pad, W1, v1, W2, v2, Wcat, bcat):
    n = ge_pad.shape[0]
    blk = 1024
    grid = n // blk
    return pl.pallas_call(
        _cell_body,
        grid=(grid,),
        in_specs=[
            pl.BlockSpec((blk, 512), lambda i: (i, 0)),
            pl.BlockSpec((512, 1024), lambda i: (0, 0)),
            pl.BlockSpec((3, 1024), lambda i: (0, 0)),
            pl.BlockSpec((1024, 256), lambda i: (0, 0)),
            pl.BlockSpec((3, 256), lambda i: (0, 0)),
            pl.BlockSpec((256, 1280), lambda i: (0, 0)),
            pl.BlockSpec((1, 1280), lambda i: (0, 0)),
        ],
        out_specs=[
            pl.BlockSpec((blk, 256), lambda i: (i, 0)),
            pl.BlockSpec((blk, 1280), lambda i: (i, 0)),
        ],
        out_shape=[
            jax.ShapeDtypeStruct((n, 256), jnp.float32),
            jax.ShapeDtypeStruct((n, 1280), jnp.float32),
        ],
    )(ge_pad, W1, v1, W2, v2, Wcat, bcat)


def _conv_body(x_ref, w1_ref, w2_ref, v_ref, out_ref):
    x = x_ref[:, 0]
    chans1 = []
    for o in range(2):
        y = (w1_ref[o, 0] * x[0:879] + w1_ref[o, 1] * x[1:880]
             + w1_ref[o, 2] * x[2:881] + w1_ref[o, 3])
        y = jnp.maximum(y, 0.0)
        y = y.reshape(293, 3, 8, 128).max(axis=1)
        chans1.append(y)
    chans2 = []
    for o in range(4):
        z = w2_ref[o, 6]
        for i in range(2):
            z = z + (w2_ref[o, 3 * i] * chans1[i][0:291]
                     + w2_ref[o, 3 * i + 1] * chans1[i][1:292]
                     + w2_ref[o, 3 * i + 2] * chans1[i][2:293])
        z = jnp.maximum(z, 0.0) * v_ref[o, 0] + v_ref[o, 1]
        z = z.reshape(97, 3, 8, 128).max(axis=1)
        chans2.append(z)
    out_ref[...] = jnp.concatenate(chans2, axis=0)[:, None]


def _drug_conv(fpT3, w1s, w2s, v2s):
    return pl.pallas_call(
        _conv_body,
        grid=(5,),
        in_specs=[
            pl.BlockSpec((881, 1, 8, 128), lambda i: (0, i, 0, 0)),
            pl.BlockSpec(memory_space=pltpu.SMEM),
            pl.BlockSpec(memory_space=pltpu.SMEM),
            pl.BlockSpec(memory_space=pltpu.SMEM),
        ],
        out_specs=pl.BlockSpec((388, 1, 8, 128), lambda i: (0, i, 0, 0)),
        out_shape=jax.ShapeDtypeStruct((388, 5, 8, 128), jnp.float32),
    )(fpT3, w1s, w2s, v2s)


def _drugfc_body(x_ref, W1_ref, v1_ref, W2_ref, v2_ref, Wc_ref, bc_ref,
                 drug_ref, proj_ref):
    x = x_ref[...]
    h = jax.lax.dot_general(x, W1_ref[...], (((0,), (0,)), ((), ())),
                            preferred_element_type=jnp.float32)
    h = jnp.maximum(h + v1_ref[0:1, :], 0.0) * v1_ref[1:2, :] + v1_ref[2:3, :]
    d = jnp.dot(h, W2_ref[...], preferred_element_type=jnp.float32)
    d = jnp.maximum(d + v2_ref[0:1, :], 0.0) * v2_ref[1:2, :] + v2_ref[2:3, :]
    drug_ref[...] = d
    proj_ref[...] = jnp.dot(d, Wc_ref[...], preferred_element_type=jnp.float32) + bc_ref[...]


def _drug_fc(x388, W1, v1, W2, v2, Wcat, bcat):
    n = x388.shape[1]
    blk = 1024
    return pl.pallas_call(
        _drugfc_body,
        grid=(n // blk,),
        in_specs=[
            pl.BlockSpec((388, blk), lambda i: (0, i)),
            pl.BlockSpec((388, 800), lambda i: (0, 0)),
            pl.BlockSpec((3, 800), lambda i: (0, 0)),
            pl.BlockSpec((800, 256), lambda i: (0, 0)),
            pl.BlockSpec((3, 256), lambda i: (0, 0)),
            pl.BlockSpec((256, 768), lambda i: (0, 0)),
            pl.BlockSpec((1, 768), lambda i: (0, 0)),
        ],
        out_specs=[
            pl.BlockSpec((blk, 256), lambda i: (i, 0)),
            pl.BlockSpec((blk, 768), lambda i: (i, 0)),
        ],
        out_shape=[
            jax.ShapeDtypeStruct((n, 256), jnp.float32),
            jax.ShapeDtypeStruct((n, 768), jnp.float32),
        ],
    )(x388, W1, v1, W2, v2, Wcat, bcat)


def _outproj_body(p_ref, s_ref, base_ref, R_ref, Wo_ref, bo_ref, a_ref, out_ref):
    ps = p_ref[0] + p_ref[1]
    den = jnp.dot(s_ref[0] + s_ref[1], R_ref[...],
                  preferred_element_type=jnp.float32) + 1e-16
    agg = ps / den
    g = agg * 0.5 * (1.0 + jax.lax.erf(agg * (1.0 / math.sqrt(2.0))))
    o = jnp.dot(g, Wo_ref[...], preferred_element_type=jnp.float32) + bo_ref[...]
    a = a_ref[0]
    out_ref[...] = a * o + (1.0 - a) * base_ref[...]


def _outproj(parts, s_parts, base, R, Wo, bo, a_vec):
    n = parts.shape[1]
    return pl.pallas_call(
        _outproj_body,
        grid=(1,),
        in_specs=[
            pl.BlockSpec((2, n, 256), lambda i: (0, 0, 0)),
            pl.BlockSpec((2, n, 16), lambda i: (0, 0, 0)),
            pl.BlockSpec((n, 256), lambda i: (0, 0)),
            pl.BlockSpec((16, 256), lambda i: (0, 0)),
            pl.BlockSpec((256, 256), lambda i: (0, 0)),
            pl.BlockSpec((1, 256), lambda i: (0, 0)),
            pl.BlockSpec(memory_space=pltpu.SMEM),
        ],
        out_specs=pl.BlockSpec((n, 256), lambda i: (0, 0)),
        out_shape=jax.ShapeDtypeStruct((n, 256), jnp.float32),
    )(parts, s_parts, base, R, Wo, bo, a_vec)


def _pair_body(L_ref, Rr_ref, W1t_ref, W1b_ref, b1_ref, W2_ref, b2_ref,
               W3_ref, b3_ref, out_ref):
    h = (jnp.dot(L_ref[...], W1t_ref[...], preferred_element_type=jnp.float32)
         + jnp.dot(Rr_ref[...], W1b_ref[...], preferred_element_type=jnp.float32)
         + b1_ref[...])
    h = jnp.maximum(h, 0.0)
    h = jnp.maximum(jnp.dot(h, W2_ref[...], preferred_element_type=jnp.float32)
                    + b2_ref[...], 0.0)
    z = jnp.dot(h, W3_ref[...], preferred_element_type=jnp.float32) + b3_ref[...]
    out_ref[...] = jax.nn.sigmoid(z)


def _pair_mlp(L, Rg, W1t, W1b, b1, W2, b2, W3p, b3p):
    n = L.shape[0]
    blk = 1024
    return pl.pallas_call(
        _pair_body,
        grid=(n // blk,),
        in_specs=[
            pl.BlockSpec((blk, 256), lambda i: (i, 0)),
            pl.BlockSpec((blk, 256), lambda i: (i, 0)),
            pl.BlockSpec((256, 256), lambda i: (0, 0)),
            pl.BlockSpec((256, 256), lambda i: (0, 0)),
            pl.BlockSpec((1, 256), lambda i: (0, 0)),
            pl.BlockSpec((256, 128), lambda i: (0, 0)),
            pl.BlockSpec((1, 128), lambda i: (0, 0)),
            pl.BlockSpec((128, 128), lambda i: (0, 0)),
            pl.BlockSpec((1, 128), lambda i: (0, 0)),
        ],
        out_specs=pl.BlockSpec((blk, 128), lambda i: (i, 0)),
        out_shape=jax.ShapeDtypeStruct((n, 128), jnp.float32),
    )(L, Rg, W1t, W1b, b1, W2, b2, W3p, b3p)


def _headT(w):
    if w.ndim == 1:
        return w.reshape(H, Dh).T.reshape(256)
    return w.reshape(-1, H, Dh).transpose(0, 2, 1).reshape(-1, 256)


def _fuse_tables(q, xside):
    t = 'c' if xside == 'c' else 'd'
    Wq, bq = q['Wq_' + t], q['bq_' + t]
    Wk, bk = q['Wk_' + t], q['bk_' + t]
    Wv, bv = q['Wv_' + t], q['bv_' + t]
    cols_w = [_headT(Wq)]
    cols_b = [_headT(bq)]
    etypes = ['cc', 'cd'] if xside == 'c' else ['dd']
    for e in etypes:
        scale = (q['p_' + e] * _SC)[None, :, None]
        Wkt = jnp.einsum('ihd,hde->ihe', Wk.reshape(256, H, Dh), q['a_' + e]) * scale
        bkt = jnp.einsum('hd,hde->he', bk.reshape(H, Dh), q['a_' + e]) * scale[0]
        Wvt = jnp.einsum('ihd,hde->ihe', Wv.reshape(256, H, Dh), q['m_' + e])
        bvt = jnp.einsum('hd,hde->he', bv.reshape(H, Dh), q['m_' + e])
        cols_w += [Wkt.transpose(0, 2, 1).reshape(256, 256),
                   Wvt.transpose(0, 2, 1).reshape(256, 256)]
        cols_b += [bkt.T.reshape(256), bvt.T.reshape(256)]
    return jnp.concatenate(cols_w, axis=1), jnp.concatenate(cols_b)[None, :]


def _edge_jnp(qt, ktt, vtt, src, dst, num):
    lg = (qt[dst].reshape(-1, Dh, H) * ktt[src].reshape(-1, Dh, H)).sum(axis=1)
    e = jnp.exp(lg)
    s = jax.ops.segment_sum(e, dst, num_segments=num)
    msg = vtt[src].reshape(-1, Dh, H) * e[:, None, :]
    agg = jax.ops.segment_sum(msg.reshape(-1, 256), dst, num_segments=num)
    return agg, s


def _gather_sc_body(xc_ref, xd_ref, ci_ref, di_ref, L_out, R_out,
                    idxv, rows):
    cid = lax.axis_index("c")
    sid = lax.axis_index("s")
    wid = sid * 2 + cid
    for k in range(2):
        base = pl.multiple_of(wid * 256 + k * 128, 128)
        pltpu.sync_copy(ci_ref.at[pl.ds(base, 128)], idxv)
        pltpu.sync_copy(xc_ref.at[idxv], rows)
        pltpu.sync_copy(rows, L_out.at[pl.ds(base, 128)])
        pltpu.sync_copy(di_ref.at[pl.ds(base, 128)], idxv)
        pltpu.sync_copy(xd_ref.at[idxv], rows)
        pltpu.sync_copy(rows, R_out.at[pl.ds(base, 128)])


def _gather_sc(xc, xd, ci, di):
    f32 = jnp.float32
    mesh = plsc.VectorSubcoreMesh(core_axis_name="c", subcore_axis_name="s",
                                  num_cores=2, num_subcores=16)
    fn = pl.kernel(
        _gather_sc_body,
        out_type=[
            jax.ShapeDtypeStruct((8192, 256), f32),
            jax.ShapeDtypeStruct((8192, 256), f32),
        ],
        mesh=mesh,
        scratch_types=[
            pltpu.VMEM((128,), jnp.int32),
            pltpu.VMEM((128, 256), f32),
        ],
    )
    return fn(xc, xd, ci, di)


def kernel(gene_expression_feature, CNV_feature, cell_edge_idx,
           fingerprint_feature, drug_edge_index, train_pair_mask,
           cell_drug_edge_idx, params):
    p = params
    q = p['hgt'][1]
    f32 = jnp.float32

    s1 = (p['cls_g1'] * _BNS)[None, :]
    v1 = jnp.concatenate([p['cls_b1'][None, :], s1, p['cls_bt1'][None, :]], 0)
    s2 = (p['cls_g2'] * _BNS)[None, :]
    v2 = jnp.concatenate([p['cls_b2'][None, :], s2, p['cls_bt2'][None, :]], 0)
    Wcat_c, bcat_c = _fuse_tables(q, 'c')
    Wcat_d, bcat_d = _fuse_tables(q, 'd')

    s1c = p['fp_g1'] * _BNS
    t1c = p['fp_bt1']
    w1s = jnp.concatenate([p['fp_cw1'][:, 0, :], p['fp_cb1'][:, None]], 1)
    w2f = p['fp_cw2'] * s1c[None, :, None]
    b2f = p['fp_cb2'] + (p['fp_cw2'] * t1c[None, :, None]).sum(axis=(1, 2))
    w2s = jnp.concatenate([w2f.reshape(4, 6), b2f[:, None]], 1)
    v2s = jnp.stack([p['fp_g2'] * _BNS, p['fp_bt2']], axis=1)
    s3 = (p['fp_g3'] * _BNS)[None, :]
    v3 = jnp.concatenate([p['fp_b1'][None, :], s3, p['fp_bt3'][None, :]], 0)
    s4 = (p['fp_g4'] * _BNS)[None, :]
    v4 = jnp.concatenate([p['fp_b2'][None, :], s4, p['fp_bt4'][None, :]], 0)

    Wo_c = _headT(q['Wo_c'].T).T
    Wo_d = _headT(q['Wo_d'].T).T
    Rrep = jnp.tile(jnp.eye(16, dtype=f32), (1, 16))
    ac = jax.nn.sigmoid(q['skip_c'])[None]
    ad = jax.nn.sigmoid(q['skip_d'])[None]

    ge_pad = jnp.pad(gene_expression_feature, ((0, NROW - 5000), (0, 0)))
    cell, projc = _cell_encoder(ge_pad, p['cls_W1'], v1, p['cls_W2'], v2,
                                Wcat_c, bcat_c)

    fpT = jnp.pad(fingerprint_feature, ((0, NROW - 5000), (0, 0))).T
    fpT3 = fpT.reshape(881, 5, 8, 128)
    x388 = _drug_conv(fpT3, w1s, w2s, v2s).reshape(388, NROW)
    drug, projd = _drug_fc(x388, p['fp_W1'], v3, p['fp_W2'], v4, Wcat_d, bcat_d)

    qc_t, ktcc, vtcc, ktcd, vtcd = [projc[:, i * 256:(i + 1) * 256] for i in range(5)]
    qd_t, ktdd, vtdd = [projd[:, i * 256:(i + 1) * 256] for i in range(3)]

    aggc, s_c = _edge_jnp(qc_t, ktcc, vtcc, cell_edge_idx[0], cell_edge_idx[1], NPAD)
    aggd1, s_d1 = _edge_jnp(qd_t, ktdd, vtdd, drug_edge_index[0], drug_edge_index[1], NPAD)
    aggd2, s_d2 = _edge_jnp(qd_t, ktcd, vtcd, cell_drug_edge_idx[0], cell_drug_edge_idx[1], NPAD)
    z256 = jnp.zeros((NPAD, 256), f32)
    z16 = jnp.zeros((NPAD, 16), f32)
    partc = jnp.stack([aggc, z256])
    spartc = jnp.stack([s_c, z16])
    partd = jnp.stack([aggd1, aggd2])
    spartd = jnp.stack([s_d1, s_d2])

    xc = _outproj(partc, spartc, cell, Rrep, Wo_c, q['bo_c'][None, :], ac)
    xd = _outproj(partd, spartd, drug, Rrep, Wo_d, q['bo_d'][None, :], ad)

    ci = train_pair_mask[:, 1]
    di = train_pair_mask[:, 0]
    L, Rg = _gather_sc(xc, xd, ci, di)

    W3p = jnp.pad(p['cmb_W3'], ((0, 0), (0, 127)))
    b3p = jnp.pad(p['cmb_b3'], (0, 127))[None, :]
    res = _pair_mlp(L, Rg, p['cmb_W1'][:256], p['cmb_W1'][256:],
                    p['cmb_b1'][None, :], p['cmb_W2'], p['cmb_b2'][None, :],
                    W3p, b3p)
    return res[:, 0]
